# baseline (speedup 1.0000x reference)
import jax
import jax.numpy as jnp
from jax import lax
from jax.experimental import pallas as pl
from jax.experimental.pallas import tpu as pltpu
from jax.experimental.pallas import tpu_sc as plsc

D_MODEL = 64
CHUNK = 128
NUM_WORKERS = 32


def _emb_body(x_hbm, tok_hbm, pos_hbm, out_hbm, idx_v, pos_v, rows_v, sem):
    wid = lax.axis_index("s") * 2 + lax.axis_index("c")
    nchunk = x_hbm.shape[1]
    seq_len = pos_v.shape[0]
    rows_per_tile = nchunk * CHUNK
    base = wid * rows_per_tile

    pltpu.sync_copy(x_hbm.at[wid], idx_v)
    pltpu.sync_copy(pos_hbm, pos_v)

    def chunk_body(g, carry):
        pltpu.async_copy(tok_hbm.at[idx_v.at[g]], rows_v, sem).wait()

        def row_body(r, c2):
            s = lax.rem(g * CHUNK + r, seq_len)
            for j in range(D_MODEL // 16):
                sl = pl.ds(16 * j, 16)
                rows_v[r, sl] = rows_v[r, sl] + pos_v[s, sl]
            return c2

        lax.fori_loop(0, CHUNK, row_body, 0)
        pltpu.sync_copy(rows_v, out_hbm.at[pl.ds(base + g * CHUNK, CHUNK)])
        return carry

    lax.fori_loop(0, nchunk, chunk_body, 0)


def kernel(x, token_embed, pos_embed):
    batch, seq_len = x.shape
    n = batch * seq_len
    nchunk = n // (NUM_WORKERS * CHUNK)
    x3 = x.reshape(NUM_WORKERS, nchunk, CHUNK).astype(jnp.int32)
    mesh = plsc.VectorSubcoreMesh(core_axis_name="c", subcore_axis_name="s")
    out = pl.kernel(
        _emb_body,
        out_type=jax.ShapeDtypeStruct((n, D_MODEL), jnp.float32),
        mesh=mesh,
        compiler_params=pltpu.CompilerParams(use_tc_tiling_on_sc=False),
        scratch_types=[
            pltpu.VMEM((nchunk, CHUNK), jnp.int32),
            pltpu.VMEM((seq_len, D_MODEL), jnp.float32),
            pltpu.VMEM((CHUNK, D_MODEL), jnp.float32),
            pltpu.SemaphoreType.DMA,
        ],
    )(x3, token_embed, pos_embed[:seq_len])
    return out.reshape(batch, seq_len, D_MODEL)


# speedup vs baseline: 1.4334x; 1.4334x over previous
import jax
import jax.numpy as jnp
from jax import lax
from jax.experimental import pallas as pl
from jax.experimental.pallas import tpu as pltpu
from jax.experimental.pallas import tpu_sc as plsc

D_MODEL = 64
NUM_WORKERS = 32


def _emb_body(x_hbm, tok_hbm, pos_hbm, out_hbm,
              idx_v, pos_v, rows0, rows1, gs0, gs1, os0, os1):
    wid = lax.axis_index("s") * 2 + lax.axis_index("c")
    nchunk = x_hbm.shape[1]
    chunk = x_hbm.shape[2]
    niter = nchunk // 2
    base = wid * nchunk * chunk

    pltpu.sync_copy(x_hbm.at[wid], idx_v)
    pltpu.sync_copy(pos_hbm, pos_v)

    def add_pos(buf):
        def row_body(r, c):
            for j in range(D_MODEL // 16):
                sl = pl.ds(16 * j, 16)
                buf[r, sl] = buf[r, sl] + pos_v[r, sl]
                buf[r + 1, sl] = buf[r + 1, sl] + pos_v[r + 1, sl]
            return c
        lax.fori_loop(0, chunk // 2, lambda r, c: row_body(2 * r, c), 0)

    def wait_gather(buf, sem):
        pltpu.make_async_copy(tok_hbm.at[pl.ds(0, chunk)], buf, sem).wait()

    def wait_out(buf, sem):
        pltpu.make_async_copy(buf, out_hbm.at[pl.ds(0, chunk)], sem).wait()

    pltpu.async_copy(tok_hbm.at[idx_v.at[0]], rows0, gs0)

    def body(i, carry):
        g0 = 2 * i
        g1 = g0 + 1
        wait_gather(rows0, gs0)
        @pl.when(i > 0)
        def _():
            wait_out(rows1, os1)
        pltpu.async_copy(tok_hbm.at[idx_v.at[g1]], rows1, gs1)
        add_pos(rows0)
        pltpu.async_copy(rows0, out_hbm.at[pl.ds(base + g0 * chunk, chunk)], os0)
        wait_gather(rows1, gs1)
        @pl.when(i < niter - 1)
        def _():
            wait_out(rows0, os0)
            pltpu.async_copy(tok_hbm.at[idx_v.at[g0 + 2]], rows0, gs0)
        add_pos(rows1)
        pltpu.async_copy(rows1, out_hbm.at[pl.ds(base + g1 * chunk, chunk)], os1)
        return carry

    lax.fori_loop(0, niter, body, 0)
    wait_out(rows0, os0)
    wait_out(rows1, os1)


def kernel(x, token_embed, pos_embed):
    batch, seq_len = x.shape
    n = batch * seq_len
    nchunk = n // (NUM_WORKERS * seq_len)
    x3 = x.reshape(NUM_WORKERS, nchunk, seq_len).astype(jnp.int32)
    mesh = plsc.VectorSubcoreMesh(core_axis_name="c", subcore_axis_name="s")
    out = pl.kernel(
        _emb_body,
        out_type=jax.ShapeDtypeStruct((n, D_MODEL), jnp.float32),
        mesh=mesh,
        compiler_params=pltpu.CompilerParams(use_tc_tiling_on_sc=False),
        scratch_types=[
            pltpu.VMEM((nchunk, seq_len), jnp.int32),
            pltpu.VMEM((seq_len, D_MODEL), jnp.float32),
            pltpu.VMEM((seq_len, D_MODEL), jnp.float32),
            pltpu.VMEM((seq_len, D_MODEL), jnp.float32),
            pltpu.SemaphoreType.DMA,
            pltpu.SemaphoreType.DMA,
            pltpu.SemaphoreType.DMA,
            pltpu.SemaphoreType.DMA,
        ],
    )(x3, token_embed, pos_embed[:seq_len])
    return out.reshape(batch, seq_len, D_MODEL)
